# Initial kernel scaffold; baseline (speedup 1.0000x reference)
import jax
import jax.numpy as jnp
from jax.experimental import pallas as pl


def kernel(lr_lv3, refsr_lv3, ref_lv3, ref_lv2, ref_lv1):
    raise NotImplementedError("write your pallas kernel here")


# speedup vs baseline: 65.9671x; 65.9671x over previous
import functools

import jax
import jax.numpy as jnp
import numpy as np
from jax import lax
from jax.experimental import pallas as pl
from jax.experimental.pallas import tpu as pltpu
from jax.experimental.pallas import tpu_sc as plsc

KS3, ST3, PD3, TOPK = 3, 1, 1, 2
L = 1600
D = 2304
PB = 400
NP = L // PB
NEG = float("-inf")
BIG = 2 ** 30
NW = 32

LEVELS = ((4, 160, 168 * 42, 64), (2, 80, 84 * 42, 128), (1, 40, 42 * 42, 256))


def _unfold(x, k, stride, padding):
    B, C, H, W = x.shape
    xp = jnp.pad(x, ((0, 0), (0, 0), (padding, padding), (padding, padding)))
    Hout = (H + 2 * padding - k) // stride + 1
    Wout = (W + 2 * padding - k) // stride + 1
    ii = (jnp.arange(Hout) * stride)[:, None, None, None] + jnp.arange(k)[None, None, :, None]
    jj = (jnp.arange(Wout) * stride)[None, :, None, None] + jnp.arange(k)[None, None, None, :]
    patches = xp[:, :, ii, jj]
    patches = jnp.transpose(patches, (0, 1, 4, 5, 2, 3)).reshape(B, C * k * k, Hout * Wout)
    return patches


def _mm_topk_kernel(k_ref, q_ref, soft_ref, hard_ref, v1, i1, v2, i2):
    ip = pl.program_id(1)
    r = jnp.dot(k_ref[0], q_ref[0], preferred_element_type=jnp.float32)
    ri = lax.broadcasted_iota(jnp.int32, (PB, L), 0) + ip * PB
    m1 = jnp.max(r, axis=0, keepdims=True)
    idx1 = jnp.min(jnp.where(r == m1, ri, BIG), axis=0, keepdims=True)
    masked = jnp.where(ri == idx1, NEG, r)
    m2 = jnp.max(masked, axis=0, keepdims=True)
    idx2 = jnp.min(jnp.where(masked == m2, ri, BIG), axis=0, keepdims=True)

    @pl.when(ip == 0)
    def _():
        v1[...] = m1
        i1[...] = idx1
        v2[...] = m2
        i2[...] = idx2

    @pl.when(ip != 0)
    def _():
        r1, j1, r2, j2 = v1[...], i1[...], v2[...], i2[...]
        c1 = m1 > r1
        ca = m2 > r1
        cb = m1 > r2
        v1[...] = jnp.where(c1, m1, r1)
        i1[...] = jnp.where(c1, idx1, j1)
        v2[...] = jnp.where(c1, jnp.where(ca, m2, r1), jnp.where(cb, m1, r2))
        i2[...] = jnp.where(c1, jnp.where(ca, idx2, j1), jnp.where(cb, idx1, j2))

    @pl.when(ip == NP - 1)
    def _():
        soft_ref[0, 0:1, :] = v1[...]
        soft_ref[0, 1:2, :] = v2[...]
        hard_ref[0, 0:1, :] = i1[...]
        hard_ref[0, 1:2, :] = i2[...]


def _mm_topk(Kb, Qb):
    B = Kb.shape[0]
    return pl.pallas_call(
        _mm_topk_kernel,
        grid=(B, NP),
        in_specs=[
            pl.BlockSpec((1, PB, D), lambda b, ip: (b, ip, 0)),
            pl.BlockSpec((1, D, L), lambda b, ip: (b, 0, 0)),
        ],
        out_specs=[
            pl.BlockSpec((1, 2, L), lambda b, ip: (b, 0, 0)),
            pl.BlockSpec((1, 2, L), lambda b, ip: (b, 0, 0)),
        ],
        out_shape=[
            jax.ShapeDtypeStruct((B, 2, L), jnp.float32),
            jax.ShapeDtypeStruct((B, 2, L), jnp.int32),
        ],
        scratch_shapes=[
            pltpu.VMEM((1, L), jnp.float32),
            pltpu.VMEM((1, L), jnp.int32),
            pltpu.VMEM((1, L), jnp.float32),
            pltpu.VMEM((1, L), jnp.int32),
        ],
    )(Kb, Qb)


def _idx_kernel(h1_ref, h2_ref, h3_ref, o1_ref, o2_ref, o3_ref):
    b = pl.program_id(0) % 2
    for (s, Ho, tabrows, _), href, oref in zip(LEVELS, (h1_ref, h2_ref, h3_ref),
                                               (o1_ref, o2_ref, o3_ref)):
        for a in range(3):
            for j in range(3):
                hq = href[0, s * (3 - a):s * (3 - a) + Ho, 2 - j:2 - j + 40]
                iy = hq // 40
                ix = hq - iy * 40
                yi = lax.broadcasted_iota(jnp.int32, (Ho, 40), 0)
                gi = lax.broadcasted_iota(jnp.int32, (Ho, 40), 1)
                ymod = yi - (yi // s) * s
                rowid = (s * (iy + a) + ymod) * 42 + (ix + j) + b * tabrows
                qyv = yi // s + (1 - a)
                qxv = gi + (1 - j)
                valid = ((qyv >= 0) & (qyv <= 39) & (qxv >= 0) & (qxv <= 39))
                oref[0, a * 3 + j] = jnp.where(valid, rowid, 0)


def _build_idx(hard):
    hkb = jnp.transpose(hard, (1, 0, 2)).reshape(4, 40, 40)
    hpad = jnp.pad(hkb, ((0, 0), (2, 2), (1, 1)))
    hreps = [jnp.repeat(hpad, s, axis=1) for s, _, _, _ in LEVELS]
    outs = pl.pallas_call(
        _idx_kernel,
        grid=(4,),
        in_specs=[pl.BlockSpec((1, 44 * s, 42), lambda kb: (kb, 0, 0))
                  for s, _, _, _ in LEVELS],
        out_specs=[pl.BlockSpec((1, 9, Ho, 40), lambda kb: (kb, 0, 0, 0))
                   for _, Ho, _, _ in LEVELS],
        out_shape=[jax.ShapeDtypeStruct((4, 9, Ho, 40), jnp.int32)
                   for _, Ho, _, _ in LEVELS],
    )(*hreps)
    return [o.transpose(0, 2, 3, 1).reshape(4, lv[1], 3, 120)
            for o, lv in zip(outs, LEVELS)]


def _sc_body(tab1, tab2, tab3, idx1, idx2, idx3, inv1, inv2, inv3,
             out1, out2, out3, idxv, buf, outv, invv, sem):
    wid = lax.axis_index("s") * 2 + lax.axis_index("c")
    for (s, Ho, tabrows, C), tabr, idxr, invr, outr in zip(
            LEVELS, (tab1, tab2, tab3), (idx1, idx2, idx3),
            (inv1, inv2, inv3), (out1, out2, out3)):
        per_w = 4 * Ho // NW

        def body(i, _, tabr=tabr, idxr=idxr, invr=invr, outr=outr, Ho=Ho):
            t = i * NW + wid
            kb = t // Ho
            Y = t - kb * Ho
            pltpu.sync_copy(idxr.at[kb, Y], idxv)
            h0 = pltpu.async_copy(tabr.at[idxv.at[0]], buf.at[pl.ds(0, 120)], sem)
            h1 = pltpu.async_copy(tabr.at[idxv.at[1]], buf.at[pl.ds(120, 120)], sem)
            h2 = pltpu.async_copy(tabr.at[idxv.at[2]], buf.at[pl.ds(240, 120)], sem)
            pltpu.sync_copy(invr.at[Y], invv)
            h0.wait()
            h1.wait()
            h2.wait()

            def gbody(g, _):
                iv = invv[pl.ds(g * 16, 16)]
                for c in range(16):
                    acc = buf[g * 9, pl.ds(c * 16, 16)]
                    for rr in range(1, 9):
                        acc = acc + buf[g * 9 + rr, pl.ds(c * 16, 16)]
                    outv[g, pl.ds(c * 16, 16)] = acc * iv
                return 0

            lax.fori_loop(0, 40, gbody, 0)
            pltpu.sync_copy(outv, outr.at[kb, Y])
            return 0

        lax.fori_loop(0, per_w, body, 0)


def _sc_gather(tabs, idxs, invs):
    mesh = plsc.VectorSubcoreMesh(core_axis_name="c", subcore_axis_name="s")
    k = functools.partial(
        pl.kernel,
        mesh=mesh,
        out_type=[jax.ShapeDtypeStruct((4, Ho, 40, 256), jnp.float32)
                  for _, Ho, _, _ in LEVELS],
        scratch_types=[
            pltpu.VMEM((3, 120), jnp.int32),
            pltpu.VMEM((360, 256), jnp.float32),
            pltpu.VMEM((40, 256), jnp.float32),
            pltpu.VMEM((640,), jnp.float32),
            pltpu.SemaphoreType.DMA,
        ],
    )(_sc_body)
    return k(*tabs, *idxs, *invs)


def _mk_tab(x, pad, px):
    B, C, H, W = x.shape
    xp = jnp.pad(x, ((0, 0), (0, 0), (pad, pad), (pad, pad)))
    cl = jnp.transpose(xp, (0, 2, 3, 1))
    Hp, Wp = H + 2 * pad, W + 2 * pad
    return cl.reshape(B * Hp * (Wp // px), px * C)


def _mk_inv(s, Ho):
    cy = np.array([min(2, Y // s + 1) - max(0, Y // s - 38) + 1 for Y in range(Ho)])
    cx = np.array([min(2, g + 1) - max(0, g - 38) + 1 for g in range(40)])
    inv = (1.0 / np.outer(cy, cx)).astype(np.float32)
    return jnp.asarray(np.repeat(inv, 16, axis=1))


def kernel(lr_lv3, refsr_lv3, ref_lv3, ref_lv2, ref_lv1):
    B, C3, h, w = lr_lv3.shape
    Q = _unfold(lr_lv3, KS3, PD3, ST3)
    K = _unfold(refsr_lv3, KS3, PD3, ST3)
    Kt = jnp.transpose(K, (0, 2, 1))
    nk = jnp.linalg.norm(Kt, axis=2, keepdims=True)
    nq = jnp.linalg.norm(Q, axis=1, keepdims=True)
    Kb = (Kt / jnp.maximum(nk, 1e-12)).astype(jnp.bfloat16)
    Qb = (Q / jnp.maximum(nq, 1e-12)).astype(jnp.bfloat16)
    soft, hard = _mm_topk(Kb, Qb)

    idxs = _build_idx(hard)
    tabs = (_mk_tab(ref_lv1, 4, 4), _mk_tab(ref_lv2, 2, 2), _mk_tab(ref_lv3, 1, 1))
    invs = tuple(_mk_inv(s, Ho) for s, Ho, _, _ in LEVELS)
    out1, out2, out3 = _sc_gather(tabs, idxs, invs)

    S = jnp.transpose(soft, (1, 0, 2)).reshape(TOPK, B, 1, h, w)
    T1 = out1.reshape(2, 2, 160, 160, 64).transpose(0, 1, 4, 2, 3)
    T2 = out2.reshape(2, 2, 80, 80, 128).transpose(0, 1, 4, 2, 3)
    T3 = out3.reshape(2, 2, 40, 40, 256).transpose(0, 1, 4, 2, 3)
    return (S, T3, T2, T1)
